# baseline (speedup 1.0000x reference)
import jax
import jax.numpy as jnp
from jax.experimental import pallas as pl
from jax.experimental.pallas import tpu as pltpu

_STRIDE = 4.0
_NMS_THRESH = 0.7
_PRE = 2000
_POST = 1000
_BIG_NEG = -1e9
_ROWS, _LANES = 16, 128
_PAD = _ROWS * _LANES


def _make_nms_body(num_images):
    def _nms_body(piv_ref, data_ref, out_ref):
        bx1 = []
        by1 = []
        bx2 = []
        by2 = []
        area = []
        for b in range(num_images):
            cx = data_ref[b, 0]
            cy = data_ref[b, 1]
            x1 = cx + data_ref[b, 2] * _STRIDE
            y1 = cy + data_ref[b, 3] * _STRIDE
            x2 = cx + data_ref[b, 4] * _STRIDE
            y2 = cy + data_ref[b, 5] * _STRIDE
            bx1.append(jnp.minimum(x1, x2))
            bx2.append(jnp.maximum(x1, x2))
            by1.append(jnp.minimum(y1, y2))
            by2.append(jnp.maximum(y1, y2))
            area.append(jnp.maximum(bx2[b] - bx1[b], 0.0)
                        * jnp.maximum(by2[b] - by1[b], 0.0))
        pos = (jax.lax.broadcasted_iota(jnp.int32, (_ROWS, _LANES), 0) * _LANES
               + jax.lax.broadcasted_iota(jnp.int32, (_ROWS, _LANES), 1))

        def body(i, keeps):
            new_keeps = []
            later = pos > i
            for b in range(num_images):
                keep = keeps[b]
                xi1 = piv_ref[b, 0, i]
                yi1 = piv_ref[b, 1, i]
                xi2 = piv_ref[b, 2, i]
                yi2 = piv_ref[b, 3, i]
                ai = piv_ref[b, 4, i]
                ki = jnp.sum(jnp.where(pos == i, keep, 0.0)) > 0.0
                xx1 = jnp.maximum(bx1[b], xi1)
                yy1 = jnp.maximum(by1[b], yi1)
                xx2 = jnp.minimum(bx2[b], xi2)
                yy2 = jnp.minimum(by2[b], yi2)
                inter = (jnp.maximum(xx2 - xx1, 0.0)
                         * jnp.maximum(yy2 - yy1, 0.0))
                union = area[b] + ai - inter
                supp = (inter > _NMS_THRESH * jnp.maximum(union, 1e-6)) & later
                new_keeps.append(jnp.where(supp & ki, 0.0, keep))
            return tuple(new_keeps)

        keep0 = (pos < _PRE).astype(jnp.float32)
        keeps = jax.lax.fori_loop(0, _PRE, body,
                                  tuple(keep0 for _ in range(num_images)))
        for b in range(num_images):
            out_ref[b, 0] = bx1[b]
            out_ref[b, 1] = by1[b]
            out_ref[b, 2] = bx2[b]
            out_ref[b, 3] = by2[b]
            out_ref[b, 4] = keeps[b]
    return _nms_body


@jax.jit
def kernel(pred_objectness_logits, pred_deltas):
    B, _, H, W = pred_objectness_logits.shape
    HW = H * W
    logits = pred_objectness_logits.reshape(B, HW)
    top_scores, top_idx = jax.lax.top_k(logits, _PRE)
    dflat = pred_deltas.reshape(B, 4, HW)
    d = jnp.take_along_axis(dflat, top_idx[:, None, :], axis=2)
    cx = (top_idx % W).astype(jnp.float32) * _STRIDE
    cy = (top_idx // W).astype(jnp.float32) * _STRIDE
    data = jnp.concatenate([cx[:, None, :], cy[:, None, :], d], axis=1)
    data = jnp.pad(data, ((0, 0), (0, 0), (0, _PAD - _PRE)))
    data = data.reshape(B, 6, _ROWS, _LANES)

    px1 = cx + d[:, 0] * _STRIDE
    py1 = cy + d[:, 1] * _STRIDE
    px2 = cx + d[:, 2] * _STRIDE
    py2 = cy + d[:, 3] * _STRIDE
    pbx1 = jnp.minimum(px1, px2)
    pbx2 = jnp.maximum(px1, px2)
    pby1 = jnp.minimum(py1, py2)
    pby2 = jnp.maximum(py1, py2)
    parea = jnp.maximum(pbx2 - pbx1, 0.0) * jnp.maximum(pby2 - pby1, 0.0)
    piv = jnp.stack([pbx1, pby1, pbx2, pby2, parea], axis=1)

    out = pl.pallas_call(
        _make_nms_body(B),
        in_specs=[
            pl.BlockSpec(memory_space=pltpu.SMEM),
            pl.BlockSpec(memory_space=pltpu.VMEM),
        ],
        out_specs=pl.BlockSpec(memory_space=pltpu.VMEM),
        out_shape=jax.ShapeDtypeStruct((B, 5, _ROWS, _LANES), jnp.float32),
    )(piv, data)

    out = out.reshape(B, 5, _PAD)[:, :, :_PRE]
    boxes = jnp.transpose(out[:, :4, :], (0, 2, 1))
    keep = out[:, 4, :] > 0.5
    order = jnp.argsort(jnp.where(keep, 0, 1), axis=1, stable=True)
    sel = order[:, :_POST]
    kept = jnp.take_along_axis(keep, sel, axis=1)
    out_boxes = jnp.take_along_axis(boxes, sel[:, :, None], axis=1)
    out_scores = jnp.where(kept, jnp.take_along_axis(top_scores, sel, axis=1),
                           _BIG_NEG)
    return jnp.concatenate([out_boxes, out_scores[:, :, None]], axis=-1)


# speedup vs baseline: 1.3145x; 1.3145x over previous
import jax
import jax.numpy as jnp
from jax.experimental import pallas as pl
from jax.experimental.pallas import tpu as pltpu

_STRIDE = 4.0
_NMS_THRESH = 0.7
_PRE = 2000
_POST = 1000
_BIG_NEG = -1e9
_ROWS, _LANES = 16, 128
_PAD = _ROWS * _LANES
_UNROLL = 4


def _make_nms_body(num_images):
    def _nms_body(piv_ref, data_ref, out_ref):
        bx1 = []
        by1 = []
        bx2 = []
        by2 = []
        area = []
        for b in range(num_images):
            cx = data_ref[b, 0]
            cy = data_ref[b, 1]
            x1 = cx + data_ref[b, 2] * _STRIDE
            y1 = cy + data_ref[b, 3] * _STRIDE
            x2 = cx + data_ref[b, 4] * _STRIDE
            y2 = cy + data_ref[b, 5] * _STRIDE
            bx1.append(jnp.minimum(x1, x2))
            bx2.append(jnp.maximum(x1, x2))
            by1.append(jnp.minimum(y1, y2))
            by2.append(jnp.maximum(y1, y2))
            area.append(jnp.maximum(bx2[b] - bx1[b], 0.0)
                        * jnp.maximum(by2[b] - by1[b], 0.0))
        pos = (jax.lax.broadcasted_iota(jnp.int32, (_ROWS, _LANES), 0) * _LANES
               + jax.lax.broadcasted_iota(jnp.int32, (_ROWS, _LANES), 1))

        def body(g, keeps):
            i0 = g * _UNROLL
            new_keeps = []
            for b in range(num_images):
                keep = keeps[b]
                pb = [(piv_ref[b, 0, i0 + a], piv_ref[b, 1, i0 + a],
                       piv_ref[b, 2, i0 + a], piv_ref[b, 3, i0 + a],
                       piv_ref[b, 4, i0 + a]) for a in range(_UNROLL)]
                kr = [jnp.sum(jnp.where(pos == i0 + a, keep, 0.0)) > 0.0
                      for a in range(_UNROLL)]
                keff = [kr[0]]
                for a in range(1, _UNROLL):
                    k = kr[a]
                    for c in range(a):
                        sx1 = jnp.maximum(pb[c][0], pb[a][0])
                        sy1 = jnp.maximum(pb[c][1], pb[a][1])
                        sx2 = jnp.minimum(pb[c][2], pb[a][2])
                        sy2 = jnp.minimum(pb[c][3], pb[a][3])
                        sint = (jnp.maximum(sx2 - sx1, 0.0)
                                * jnp.maximum(sy2 - sy1, 0.0))
                        sun = pb[c][4] + pb[a][4] - sint
                        scond = sint > _NMS_THRESH * jnp.maximum(sun, 1e-6)
                        k = k & jnp.logical_not(keff[c] & scond)
                    keff.append(k)
                for a in range(_UNROLL):
                    xx1 = jnp.maximum(bx1[b], pb[a][0])
                    yy1 = jnp.maximum(by1[b], pb[a][1])
                    xx2 = jnp.minimum(bx2[b], pb[a][2])
                    yy2 = jnp.minimum(by2[b], pb[a][3])
                    inter = (jnp.maximum(xx2 - xx1, 0.0)
                             * jnp.maximum(yy2 - yy1, 0.0))
                    union = area[b] + pb[a][4] - inter
                    supp = ((inter > _NMS_THRESH * jnp.maximum(union, 1e-6))
                            & (pos > i0 + a))
                    keep = jnp.where(supp & keff[a], 0.0, keep)
                new_keeps.append(keep)
            return tuple(new_keeps)

        keep0 = (pos < _PRE).astype(jnp.float32)
        keeps = jax.lax.fori_loop(0, _PRE // _UNROLL, body,
                                  tuple(keep0 for _ in range(num_images)))
        for b in range(num_images):
            out_ref[b, 0] = bx1[b]
            out_ref[b, 1] = by1[b]
            out_ref[b, 2] = bx2[b]
            out_ref[b, 3] = by2[b]
            out_ref[b, 4] = keeps[b]
    return _nms_body


@jax.jit
def kernel(pred_objectness_logits, pred_deltas):
    B, _, H, W = pred_objectness_logits.shape
    HW = H * W
    logits = pred_objectness_logits.reshape(B, HW)
    top_scores, top_idx = jax.lax.top_k(logits, _PRE)
    dflat = pred_deltas.reshape(B, 4, HW)
    d = jnp.take_along_axis(dflat, top_idx[:, None, :], axis=2)
    cx = (top_idx % W).astype(jnp.float32) * _STRIDE
    cy = (top_idx // W).astype(jnp.float32) * _STRIDE
    data = jnp.concatenate([cx[:, None, :], cy[:, None, :], d], axis=1)
    data = jnp.pad(data, ((0, 0), (0, 0), (0, _PAD - _PRE)))
    data = data.reshape(B, 6, _ROWS, _LANES)

    px1 = cx + d[:, 0] * _STRIDE
    py1 = cy + d[:, 1] * _STRIDE
    px2 = cx + d[:, 2] * _STRIDE
    py2 = cy + d[:, 3] * _STRIDE
    pbx1 = jnp.minimum(px1, px2)
    pbx2 = jnp.maximum(px1, px2)
    pby1 = jnp.minimum(py1, py2)
    pby2 = jnp.maximum(py1, py2)
    parea = jnp.maximum(pbx2 - pbx1, 0.0) * jnp.maximum(pby2 - pby1, 0.0)
    piv = jnp.stack([pbx1, pby1, pbx2, pby2, parea], axis=1)

    out = pl.pallas_call(
        _make_nms_body(B),
        in_specs=[
            pl.BlockSpec(memory_space=pltpu.SMEM),
            pl.BlockSpec(memory_space=pltpu.VMEM),
        ],
        out_specs=pl.BlockSpec(memory_space=pltpu.VMEM),
        out_shape=jax.ShapeDtypeStruct((B, 5, _ROWS, _LANES), jnp.float32),
    )(piv, data)

    out = out.reshape(B, 5, _PAD)[:, :, :_PRE]
    boxes = jnp.transpose(out[:, :4, :], (0, 2, 1))
    keep = out[:, 4, :] > 0.5
    order = jnp.argsort(jnp.where(keep, 0, 1), axis=1, stable=True)
    sel = order[:, :_POST]
    kept = jnp.take_along_axis(keep, sel, axis=1)
    out_boxes = jnp.take_along_axis(boxes, sel[:, :, None], axis=1)
    out_scores = jnp.where(kept, jnp.take_along_axis(top_scores, sel, axis=1),
                           _BIG_NEG)
    return jnp.concatenate([out_boxes, out_scores[:, :, None]], axis=-1)
